# Initial kernel scaffold; baseline (speedup 1.0000x reference)
import jax
import jax.numpy as jnp
from jax.experimental import pallas as pl


def kernel(adj_indices, adj_values, embeds):
    raise NotImplementedError("write your pallas kernel here")


# speedup vs baseline: 5.9931x; 5.9931x over previous
import functools

import jax
import jax.numpy as jnp
from jax import lax
from jax.experimental import pallas as pl
from jax.experimental.pallas import tpu as pltpu, tpu_sc as plsc

N = 10000
E = 320000
D = 128
SLOPE = 0.2

NC = 2
NS = 16
NW = NC * NS
EPT = E // NW
CHUNK = 80
CPB = 25
NBLK = EPT // (CHUNK * CPB)
NP = 10240
RPT = NP // NS
LANE = 16
NQ = D // LANE

_GDN = lax.GatherDimensionNumbers(
    offset_dims=(), collapsed_slice_dims=(0,), start_index_map=(0,))


def _splat(vec, r):
    idx = jnp.full((LANE, 1), r, jnp.int32)
    return lax.gather(vec, idx, _GDN, (1,),
                      mode=lax.GatherScatterMode.PROMISE_IN_BOUNDS)


@functools.partial(
    pl.kernel,
    out_type=jax.ShapeDtypeStruct((NC, NP, D), jnp.float32),
    mesh=plsc.VectorSubcoreMesh(core_axis_name="c", subcore_axis_name="s"),
    scratch_types=[
        pltpu.VMEM((CPB, CHUNK), jnp.int32),
        pltpu.VMEM((CPB, CHUNK), jnp.int32),
        pltpu.VMEM((CPB, CHUNK), jnp.float32),
        pltpu.VMEM((CHUNK, D), jnp.float32),
        pltpu.VMEM_SHARED((NP, D), jnp.float32),
        pltpu.SemaphoreType.DMA,
    ],
)
def _spmm_sc(rows_hbm, cols_hbm, vals_hbm, embeds_hbm, out_hbm,
             rows_v, cols_v, vals_v, gbuf, acc, sem):
    c = lax.axis_index("c")
    s = lax.axis_index("s")
    wid = c * NS + s

    zvec = jnp.zeros((LANE,), jnp.float32)

    def zero_row(r, _):
        for q in range(NQ):
            gbuf[r, pl.ds(q * LANE, LANE)] = zvec
        return 0

    lax.fori_loop(0, CHUNK, zero_row, 0)
    for k in range(RPT // CHUNK):
        pltpu.sync_copy(gbuf, acc.at[pl.ds(s * RPT + k * CHUNK, CHUNK)])
    plsc.subcore_barrier()

    def block_body(b, _):
        pltpu.sync_copy(rows_hbm.at[wid, b], rows_v)
        pltpu.sync_copy(cols_hbm.at[wid, b], cols_v)
        pltpu.sync_copy(vals_hbm.at[wid, b], vals_v)

        def chunk_body(j, _):
            pltpu.async_copy(embeds_hbm.at[cols_v.at[j]], gbuf, sem).wait()

            def grp_body(g, _):
                v16 = vals_v[j, pl.ds(g * LANE, LANE)]
                for r in range(LANE):
                    splat = _splat(v16, r)
                    row = g * LANE + r
                    for q in range(NQ):
                        gbuf[row, pl.ds(q * LANE, LANE)] = (
                            gbuf[row, pl.ds(q * LANE, LANE)] * splat)
                return 0

            lax.fori_loop(0, CHUNK // LANE, grp_body, 0)
            pltpu.sync_copy(gbuf, acc.at[rows_v.at[j]], add=True)
            return 0

        lax.fori_loop(0, CPB, chunk_body, 0)
        return 0

    lax.fori_loop(0, NBLK, block_body, 0)
    plsc.subcore_barrier()

    pltpu.sync_copy(acc.at[pl.ds(s * RPT, RPT)], out_hbm.at[c, pl.ds(s * RPT, RPT)])


_BLK = 640


def _combine_body(p_ref, o_ref):
    x = p_ref[0] + p_ref[1]
    o_ref[...] = jnp.maximum(x, SLOPE * x)


def _combine(partials):
    return pl.pallas_call(
        _combine_body,
        out_shape=jax.ShapeDtypeStruct((NP, D), jnp.float32),
        grid=(NP // _BLK,),
        in_specs=[pl.BlockSpec((NC, _BLK, D), lambda i: (0, i, 0))],
        out_specs=pl.BlockSpec((_BLK, D), lambda i: (i, 0)),
    )(partials)


def kernel(adj_indices, adj_values, embeds):
    rows = adj_indices[0].astype(jnp.int32).reshape(NW, NBLK, CPB, CHUNK)
    cols = adj_indices[1].astype(jnp.int32).reshape(NW, NBLK, CPB, CHUNK)
    vals = adj_values.reshape(NW, NBLK, CPB, CHUNK)
    partials = _spmm_sc(rows, cols, vals, embeds)
    return _combine(partials)[:N]
